# Initial kernel scaffold; baseline (speedup 1.0000x reference)
import jax
import jax.numpy as jnp
from jax.experimental import pallas as pl


def kernel(h, edge_index):
    raise NotImplementedError("write your pallas kernel here")


# speedup vs baseline: 7.2298x; 7.2298x over previous
import jax
import jax.numpy as jnp
from jax import lax
from jax.experimental import pallas as pl
from jax.experimental.pallas import tpu as pltpu
from jax.experimental.pallas import tpu_sc as plsc

N_NODES = 10000
N_EDGES = 320000
D_FEAT = 128

NC = 2
NS = 16
DH = D_FEAT // NC
CHUNK = 125
ROWS_PER_TILE = N_EDGES // (NS * CHUNK)
N_PAD = 10240
NODES_PER_TILE = N_PAD // NS
FLUSH_ROWS = 128


def _sc_body(h0_hbm, h1_hbm, src_hbm, dst_hbm, partial_hbm, deg_hbm,
             src_v, dst_v, rows_v, ones_v, zrow_v, zdeg_v, acc_sh, deg_sh):
    c = lax.axis_index("c")
    s = lax.axis_index("s")

    zero16 = jnp.zeros((16,), jnp.float32)
    one16 = jnp.ones((16,), jnp.float32)

    def fill_zrow(i, carry):
        for k in range(DH // 16):
            zrow_v[i, pl.ds(k * 16, 16)] = zero16
        return carry

    lax.fori_loop(0, FLUSH_ROWS, fill_zrow, 0)

    def fill_small(i, carry):
        zdeg_v[i, :] = zero16
        return carry

    lax.fori_loop(0, FLUSH_ROWS, fill_small, 0)

    def fill_ones(i, carry):
        ones_v[i, :] = one16
        return carry

    lax.fori_loop(0, CHUNK, fill_ones, 0)

    for i in range(NODES_PER_TILE // FLUSH_ROWS):
        r0 = s * NODES_PER_TILE + i * FLUSH_ROWS
        pltpu.sync_copy(zrow_v, acc_sh.at[pl.ds(r0, FLUSH_ROWS)])

    @pl.when(c == 0)
    def _():
        for i in range(NODES_PER_TILE // FLUSH_ROWS):
            r0 = s * NODES_PER_TILE + i * FLUSH_ROWS
            pltpu.sync_copy(zdeg_v, deg_sh.at[pl.ds(r0, FLUSH_ROWS)])

    plsc.subcore_barrier()

    pltpu.sync_copy(src_hbm.at[pl.ds(s * ROWS_PER_TILE, ROWS_PER_TILE)], src_v)
    pltpu.sync_copy(dst_hbm.at[pl.ds(s * ROWS_PER_TILE, ROWS_PER_TILE)], dst_v)

    def body(j, carry):
        @pl.when(c == 0)
        def _():
            pltpu.sync_copy(h0_hbm.at[src_v.at[j]], rows_v)
            pltpu.sync_copy(ones_v, deg_sh.at[dst_v.at[j]], add=True)

        @pl.when(c == 1)
        def _():
            pltpu.sync_copy(h1_hbm.at[src_v.at[j]], rows_v)

        pltpu.sync_copy(rows_v, acc_sh.at[dst_v.at[j]], add=True)
        return carry

    lax.fori_loop(0, ROWS_PER_TILE, body, 0)

    plsc.subcore_barrier()

    for i in range(NODES_PER_TILE // FLUSH_ROWS):
        r0 = s * NODES_PER_TILE + i * FLUSH_ROWS
        pltpu.sync_copy(acc_sh.at[pl.ds(r0, FLUSH_ROWS)],
                        partial_hbm.at[pl.ds(c * N_PAD + r0, FLUSH_ROWS)])

    @pl.when(c == 0)
    def _():
        for i in range(NODES_PER_TILE // FLUSH_ROWS):
            r0 = s * NODES_PER_TILE + i * FLUSH_ROWS
            pltpu.sync_copy(deg_sh.at[pl.ds(r0, FLUSH_ROWS)],
                            deg_hbm.at[pl.ds(r0, FLUSH_ROWS)])


def _combine_body(p_ref, d_ref, o_ref):
    p = p_ref[...]
    d = d_ref[...]
    inv = 1.0 / jnp.maximum(d[:, 0:1], 1.0)
    o_ref[...] = jnp.concatenate([p[0] * inv, p[1] * inv], axis=1)


def kernel(h, edge_index):
    ei = edge_index.astype(jnp.int32)
    src = ei[0].reshape(N_EDGES // CHUNK, CHUNK)
    dst = ei[1].reshape(N_EDGES // CHUNK, CHUNK)
    h0 = h[:, :DH]
    h1 = h[:, DH:]

    mesh = plsc.VectorSubcoreMesh(core_axis_name="c", subcore_axis_name="s")

    sc_call = pl.kernel(
        _sc_body,
        mesh=mesh,
        compiler_params=pltpu.CompilerParams(use_tc_tiling_on_sc=False),
        out_type=[
            jax.ShapeDtypeStruct((NC * N_PAD, DH), jnp.float32),
            jax.ShapeDtypeStruct((N_PAD, 16), jnp.float32),
        ],
        scratch_types=[
            pltpu.VMEM((ROWS_PER_TILE, CHUNK), jnp.int32),
            pltpu.VMEM((ROWS_PER_TILE, CHUNK), jnp.int32),
            pltpu.VMEM((CHUNK, DH), jnp.float32),
            pltpu.VMEM((CHUNK, 16), jnp.float32),
            pltpu.VMEM((FLUSH_ROWS, DH), jnp.float32),
            pltpu.VMEM((FLUSH_ROWS, 16), jnp.float32),
            pltpu.VMEM_SHARED((N_PAD, DH), jnp.float32),
            pltpu.VMEM_SHARED((N_PAD, 16), jnp.float32),
        ],
    )
    partial_flat, deg = sc_call(h0, h1, src, dst)

    partial = partial_flat.reshape(NC, N_PAD, DH)

    rows_blk = 1000
    out = pl.pallas_call(
        _combine_body,
        grid=(N_NODES // rows_blk,),
        in_specs=[
            pl.BlockSpec((NC, rows_blk, DH), lambda i: (0, i, 0)),
            pl.BlockSpec((rows_blk, 16), lambda i: (i, 0)),
        ],
        out_specs=pl.BlockSpec((rows_blk, D_FEAT), lambda i: (i, 0)),
        out_shape=jax.ShapeDtypeStruct((N_NODES, D_FEAT), jnp.float32),
    )(partial, deg)
    return out


# speedup vs baseline: 13.7407x; 1.9006x over previous
import jax
import jax.numpy as jnp
from jax import lax
from jax.experimental import pallas as pl
from jax.experimental.pallas import tpu as pltpu
from jax.experimental.pallas import tpu_sc as plsc

N_NODES = 10000
N_EDGES = 320000
D_FEAT = 128

NC = 2
NS = 16
DH = D_FEAT // NC
CHUNK = 125
ROWS_PER_TILE = N_EDGES // (NS * CHUNK)
N_PAD = 10240
NODES_PER_TILE = N_PAD // NS
FLUSH_ROWS = 128
ZROWS = 32
K = 2
NG = ROWS_PER_TILE // K
DROWS = N_PAD // 16
DROWS_PER_TILE = DROWS // NS


def _sc_body(h2_hbm, srca_hbm, srcb_hbm, dst_hbm, partial_hbm, deg_hbm,
             src_v, dst_v,
             ba0, ba1, bb0, bb1,
             dloc, idx5, zrow_v, acc_sh, deg_sh,
             gsem_a, gsem_b, ssem):
    c = lax.axis_index("c")
    s = lax.axis_index("s")
    bufs_a = [ba0, ba1]
    bufs_b = [bb0, bb1]

    zero16 = jnp.zeros((16,), jnp.float32)
    one16 = jnp.ones((16,), jnp.float32)
    iota16 = lax.broadcasted_iota(jnp.int32, (16,), 0)
    tail_mask = iota16 >= (16 - (CHUNK - (CHUNK // 16) * 16))

    def fill_zrow(i, carry):
        for k in range(DH // 16):
            zrow_v[i, pl.ds(k * 16, 16)] = zero16
        return carry

    lax.fori_loop(0, ZROWS, fill_zrow, 0)

    def fill_dloc(i, carry):
        dloc[i, :] = zero16
        return carry

    lax.fori_loop(0, DROWS, fill_dloc, 0)

    for r in range(DROWS // 128):
        for k in range(8):
            idx5[r, pl.ds(16 * k, 16)] = iota16 + (128 * r + 16 * k)

    for i in range(NODES_PER_TILE // ZROWS):
        r0 = s * NODES_PER_TILE + i * ZROWS
        pltpu.sync_copy(zrow_v, acc_sh.at[pl.ds(r0, ZROWS)])
    pltpu.sync_copy(dloc.at[pl.ds(0, DROWS_PER_TILE)],
                    deg_sh.at[pl.ds(s * DROWS_PER_TILE, DROWS_PER_TILE)])

    plsc.subcore_barrier()

    @pl.when(c == 0)
    def _():
        pltpu.sync_copy(srca_hbm.at[pl.ds(s * ROWS_PER_TILE, ROWS_PER_TILE)], src_v)

    @pl.when(c == 1)
    def _():
        pltpu.sync_copy(srcb_hbm.at[pl.ds(s * ROWS_PER_TILE, ROWS_PER_TILE)], src_v)

    pltpu.sync_copy(dst_hbm.at[pl.ds(s * ROWS_PER_TILE, ROWS_PER_TILE)], dst_v)

    def hist_row(j):
        for k in range(CHUNK // 16):
            d16 = dst_v[j, pl.ds(16 * k, 16)]
            plsc.addupdate_scatter(
                dloc, [lax.shift_right_logical(d16, 4), d16 & 15], one16)
        d16 = dst_v[j, pl.ds(CHUNK - 16, 16)]
        plsc.addupdate_scatter(
            dloc, [lax.shift_right_logical(d16, 4), d16 & 15], one16,
            mask=tail_mask)

    def do_group(g, bufs, gsem, obufs, ogsem):
        @pl.when(g + 1 < NG)
        def _():
            for b in range(K):
                pltpu.async_copy(h2_hbm.at[src_v.at[(g + 1) * K + b]],
                                 obufs[b], ogsem)

        for b in range(K):
            pltpu.make_async_copy(h2_hbm.at[src_v.at[g * K + b]],
                                  bufs[b], gsem).wait()

        scat = []
        for b in range(K):
            scat.append(pltpu.async_copy(bufs[b], acc_sh.at[dst_v.at[g * K + b]],
                                         ssem, add=True))

        for b in range(K):
            hist_row(g * K + b)

        for x in scat:
            x.wait()

    for b in range(K):
        pltpu.async_copy(h2_hbm.at[src_v.at[b]], bufs_a[b], gsem_a)

    def outer(t, carry):
        do_group(2 * t, bufs_a, gsem_a, bufs_b, gsem_b)
        do_group(2 * t + 1, bufs_b, gsem_b, bufs_a, gsem_a)
        return carry

    lax.fori_loop(0, NG // 2, outer, 0)

    for r in range(DROWS // 128):
        pltpu.sync_copy(dloc.at[pl.ds(128 * r, 128)],
                        deg_sh.at[idx5.at[r]], add=True)

    plsc.subcore_barrier()

    for i in range(NODES_PER_TILE // FLUSH_ROWS):
        r0 = s * NODES_PER_TILE + i * FLUSH_ROWS
        pltpu.sync_copy(acc_sh.at[pl.ds(r0, FLUSH_ROWS)],
                        partial_hbm.at[pl.ds(c * N_PAD + r0, FLUSH_ROWS)])

    @pl.when(c == 0)
    def _():
        pltpu.sync_copy(deg_sh.at[pl.ds(s * DROWS_PER_TILE, DROWS_PER_TILE)],
                        dloc.at[pl.ds(0, DROWS_PER_TILE)])

        def expand(i, carry):
            r = DROWS_PER_TILE - 1 - i
            v = dloc[r, :]
            for lane in range(16):
                dloc[16 * r + lane, :] = jnp.broadcast_to(v[lane], (16,))
            return carry

        lax.fori_loop(0, DROWS_PER_TILE, expand, 0)
        pltpu.sync_copy(dloc, deg_hbm.at[pl.ds(s * NODES_PER_TILE, NODES_PER_TILE)])


def _combine_body(p_ref, d_ref, o_ref):
    p = p_ref[...]
    d = d_ref[...]
    inv = 1.0 / jnp.maximum(d[:, 0:1], 1.0)
    o_ref[...] = jnp.concatenate([p[0] * inv, p[1] * inv], axis=1)


def kernel(h, edge_index):
    ei = edge_index.astype(jnp.int32)
    src2 = ei[0] * 2
    srca = src2.reshape(N_EDGES // CHUNK, CHUNK)
    srcb = (src2 + 1).reshape(N_EDGES // CHUNK, CHUNK)
    dst = ei[1].reshape(N_EDGES // CHUNK, CHUNK)
    h2 = h.reshape(NC * N_NODES, DH)

    mesh = plsc.VectorSubcoreMesh(core_axis_name="c", subcore_axis_name="s")

    sc_call = pl.kernel(
        _sc_body,
        mesh=mesh,
        compiler_params=pltpu.CompilerParams(use_tc_tiling_on_sc=False,
                                             needs_layout_passes=False),
        out_type=[
            jax.ShapeDtypeStruct((NC * N_PAD, DH), jnp.float32),
            jax.ShapeDtypeStruct((N_PAD, 16), jnp.float32),
        ],
        scratch_types=[
            pltpu.VMEM((ROWS_PER_TILE, CHUNK), jnp.int32),
            pltpu.VMEM((ROWS_PER_TILE, CHUNK), jnp.int32),
        ] + [pltpu.VMEM((CHUNK, DH), jnp.float32)] * (2 * K) + [
            pltpu.VMEM((N_PAD // 16, 16), jnp.float32),
            pltpu.VMEM((N_PAD // 16 // 128, 128), jnp.int32),
            pltpu.VMEM((ZROWS, DH), jnp.float32),
            pltpu.VMEM_SHARED((N_PAD, DH), jnp.float32),
            pltpu.VMEM_SHARED((N_PAD // 16, 16), jnp.float32),
            pltpu.SemaphoreType.DMA,
            pltpu.SemaphoreType.DMA,
            pltpu.SemaphoreType.DMA,
        ],
    )
    partial_flat, deg = sc_call(h2, srca, srcb, dst)

    partial = partial_flat.reshape(NC, N_PAD, DH)

    rows_blk = 1000
    out = pl.pallas_call(
        _combine_body,
        grid=(N_NODES // rows_blk,),
        in_specs=[
            pl.BlockSpec((NC, rows_blk, DH), lambda i: (0, i, 0)),
            pl.BlockSpec((rows_blk, 16), lambda i: (i, 0)),
        ],
        out_specs=pl.BlockSpec((rows_blk, D_FEAT), lambda i: (i, 0)),
        out_shape=jax.ShapeDtypeStruct((N_NODES, D_FEAT), jnp.float32),
    )(partial, deg)
    return out


# speedup vs baseline: 16.0103x; 1.1652x over previous
import jax
import jax.numpy as jnp
from jax import lax
from jax.experimental import pallas as pl
from jax.experimental.pallas import tpu as pltpu
from jax.experimental.pallas import tpu_sc as plsc

N_NODES = 10000
N_EDGES = 320000
D_FEAT = 128

NC = 2
NS = 16
DH = D_FEAT // NC
CHUNK = 125
ROWS_PER_TILE = N_EDGES // (NS * CHUNK)
N_PAD = 10240
NODES_PER_TILE = N_PAD // NS
ZROWS = 32
K = 2
NG = ROWS_PER_TILE // K
DROWS = N_PAD // 16
DROWS_PER_TILE = DROWS // NS


def _sc_body(h2_hbm, src_hbm, dst_hbm, out_hbm,
             src_v, dst_v,
             ba0, ba1, bb0, bb1,
             dloc, idx5, idx6, idx7, zrow_v, acc_sh, deg_sh,
             gsem_a, gsem_b, ssem):
    c = lax.axis_index("c")
    s = lax.axis_index("s")
    bufs_a = [ba0, ba1]
    bufs_b = [bb0, bb1]

    zero16 = jnp.zeros((16,), jnp.float32)
    one16 = jnp.ones((16,), jnp.float32)
    iota16 = lax.broadcasted_iota(jnp.int32, (16,), 0)
    tail_mask = iota16 >= (16 - (CHUNK - (CHUNK // 16) * 16))
    cvec = jnp.broadcast_to(c, (16,))

    def fill_zrow(i, carry):
        for k in range(DH // 16):
            zrow_v[i, pl.ds(k * 16, 16)] = zero16
        return carry

    lax.fori_loop(0, ZROWS, fill_zrow, 0)

    def fill_dloc(i, carry):
        dloc[i, :] = zero16
        return carry

    lax.fori_loop(0, DROWS, fill_dloc, 0)

    for r in range(DROWS // 128):
        for k in range(8):
            idx5[r, pl.ds(16 * k, 16)] = iota16 + (128 * r + 16 * k)

    obase = 2 * NODES_PER_TILE * s + c
    for r in range(5):
        for k in range(7):
            idx6[r, pl.ds(16 * k, 16)] = 2 * iota16 + (2 * (125 * r + 16 * k)) + obase
        idx6[r, pl.ds(109, 16)] = 2 * iota16 + (2 * (125 * r + 109)) + obase
    for t, off in enumerate((624, 375, 384)):
        idx7[t, :] = 2 * iota16 + 2 * off + obase

    for i in range(NODES_PER_TILE // ZROWS):
        r0 = s * NODES_PER_TILE + i * ZROWS
        pltpu.sync_copy(zrow_v, acc_sh.at[pl.ds(r0, ZROWS)])
    pltpu.sync_copy(dloc.at[pl.ds(0, DROWS_PER_TILE)],
                    deg_sh.at[pl.ds(s * DROWS_PER_TILE, DROWS_PER_TILE)])

    plsc.subcore_barrier()

    pltpu.sync_copy(src_hbm.at[pl.ds(s * ROWS_PER_TILE, ROWS_PER_TILE)], src_v)
    pltpu.sync_copy(dst_hbm.at[pl.ds(s * ROWS_PER_TILE, ROWS_PER_TILE)], dst_v)

    def xform(i, carry):
        vals = [src_v[i, pl.ds(16 * k, 16)] for k in range(7)]
        vtail = src_v[i, pl.ds(109, 16)]
        for k in range(7):
            src_v[i, pl.ds(16 * k, 16)] = vals[k] + vals[k] + cvec
        src_v[i, pl.ds(109, 16)] = vtail + vtail + cvec
        return carry

    lax.fori_loop(0, ROWS_PER_TILE, xform, 0)

    def hist_row(j):
        for k in range(CHUNK // 16):
            d16 = dst_v[j, pl.ds(16 * k, 16)]
            plsc.addupdate_scatter(
                dloc, [lax.shift_right_logical(d16, 4), d16 & 15], one16)
        d16 = dst_v[j, pl.ds(CHUNK - 16, 16)]
        plsc.addupdate_scatter(
            dloc, [lax.shift_right_logical(d16, 4), d16 & 15], one16,
            mask=tail_mask)

    def do_group(g, bufs, gsem, obufs, ogsem):
        @pl.when(g + 1 < NG)
        def _():
            for b in range(K):
                pltpu.async_copy(h2_hbm.at[src_v.at[(g + 1) * K + b]],
                                 obufs[b], ogsem)

        for b in range(K):
            pltpu.make_async_copy(h2_hbm.at[src_v.at[g * K + b]],
                                  bufs[b], gsem).wait()

        scat = []
        for b in range(K):
            scat.append(pltpu.async_copy(bufs[b], acc_sh.at[dst_v.at[g * K + b]],
                                         ssem, add=True))

        for b in range(K):
            hist_row(g * K + b)

        for x in scat:
            x.wait()

    for b in range(K):
        pltpu.async_copy(h2_hbm.at[src_v.at[b]], bufs_a[b], gsem_a)

    def outer(t, carry):
        do_group(2 * t, bufs_a, gsem_a, bufs_b, gsem_b)
        do_group(2 * t + 1, bufs_b, gsem_b, bufs_a, gsem_a)
        return carry

    lax.fori_loop(0, NG // 2, outer, 0)

    for r in range(DROWS // 128):
        pltpu.sync_copy(dloc.at[pl.ds(128 * r, 128)],
                        deg_sh.at[idx5.at[r]], add=True)

    plsc.subcore_barrier()

    pltpu.sync_copy(deg_sh.at[pl.ds(s * DROWS_PER_TILE, DROWS_PER_TILE)],
                    dloc.at[pl.ds(0, DROWS_PER_TILE)])

    def expand(i, carry):
        r = DROWS_PER_TILE - 1 - i
        v = 1.0 / jnp.maximum(dloc[r, :], 1.0)
        for lane in range(16):
            dloc[16 * r + lane, :] = jnp.broadcast_to(v[lane], (16,))
        return carry

    lax.fori_loop(0, DROWS_PER_TILE, expand, 0)

    def mul_rows(buf, o, length):
        def mul_row(j, carry):
            iv = dloc[o + j, :]
            for k in range(DH // 16):
                buf[j, pl.ds(16 * k, 16)] = buf[j, pl.ds(16 * k, 16)] * iv
            return carry

        lax.fori_loop(0, length, mul_row, 0)

    def stage_in(buf, o, length, sem):
        return pltpu.async_copy(
            acc_sh.at[pl.ds(s * NODES_PER_TILE + o, length)],
            buf.at[pl.ds(0, length)], sem)

    def flush(chunks):
        outs = {}
        cin = stage_in(bufs_a[0], chunks[0][0], chunks[0][1], gsem_a)
        for t, (o, length, idxref, irow) in enumerate(chunks):
            buf = (bufs_a if t % 2 == 0 else bufs_b)[0]
            cin.wait()
            if t + 1 < len(chunks):
                nbuf = (bufs_a if t % 2 == 1 else bufs_b)[0]
                nsem = gsem_a if t % 2 == 1 else gsem_b
                if t - 1 in outs:
                    outs[t - 1].wait()
                cin = stage_in(nbuf, chunks[t + 1][0], chunks[t + 1][1], nsem)
            mul_rows(buf, o, length)
            outs[t] = pltpu.async_copy(buf.at[pl.ds(0, length)],
                                       out_hbm.at[idxref.at[irow]], ssem)
        for t in (len(chunks) - 2, len(chunks) - 1):
            if t in outs:
                outs[t].wait()

    @pl.when(s < NS - 1)
    def _():
        flush([(125 * r, 125, idx6, r) for r in range(5)] + [(624, 16, idx7, 0)])

    @pl.when(s == NS - 1)
    def _():
        flush([(125 * r, 125, idx6, r) for r in range(3)]
              + [(375, 16, idx7, 1), (384, 16, idx7, 2)])


def kernel(h, edge_index):
    ei = edge_index.astype(jnp.int32)
    src = ei[0].reshape(N_EDGES // CHUNK, CHUNK)
    dst = ei[1].reshape(N_EDGES // CHUNK, CHUNK)
    h2 = h.reshape(NC * N_NODES, DH)

    mesh = plsc.VectorSubcoreMesh(core_axis_name="c", subcore_axis_name="s")

    sc_call = pl.kernel(
        _sc_body,
        mesh=mesh,
        compiler_params=pltpu.CompilerParams(use_tc_tiling_on_sc=False,
                                             needs_layout_passes=False),
        out_type=[
            jax.ShapeDtypeStruct((NC * N_NODES, DH), jnp.float32),
        ],
        scratch_types=[
            pltpu.VMEM((ROWS_PER_TILE, CHUNK), jnp.int32),
            pltpu.VMEM((ROWS_PER_TILE, CHUNK), jnp.int32),
        ] + [pltpu.VMEM((CHUNK, DH), jnp.float32)] * (2 * K) + [
            pltpu.VMEM((N_PAD // 16, 16), jnp.float32),
            pltpu.VMEM((N_PAD // 16 // 128, 128), jnp.int32),
            pltpu.VMEM((5, CHUNK), jnp.int32),
            pltpu.VMEM((3, 16), jnp.int32),
            pltpu.VMEM((ZROWS, DH), jnp.float32),
            pltpu.VMEM_SHARED((N_PAD, DH), jnp.float32),
            pltpu.VMEM_SHARED((N_PAD // 16, 16), jnp.float32),
            pltpu.SemaphoreType.DMA,
            pltpu.SemaphoreType.DMA,
            pltpu.SemaphoreType.DMA,
        ],
    )
    (out2,) = sc_call(h2, src, dst)
    return out2.reshape(N_NODES, D_FEAT)


# speedup vs baseline: 16.6410x; 1.0394x over previous
import jax
import jax.numpy as jnp
from jax import lax
from jax.experimental import pallas as pl
from jax.experimental.pallas import tpu as pltpu
from jax.experimental.pallas import tpu_sc as plsc

N_NODES = 10000
N_EDGES = 320000
D_FEAT = 128

NC = 2
NS = 16
DH = D_FEAT // NC
CHUNK = 125
ROWS_PER_TILE = N_EDGES // (NS * CHUNK)
N_PAD = 10240
NODES_PER_TILE = N_PAD // NS
ZROWS = 32
K = 2
NG = ROWS_PER_TILE // K
DROWS = N_PAD // 16
DROWS_PER_TILE = DROWS // NS


def _sc_body(h2_hbm, src_hbm, dst_hbm, out_hbm,
             src_v, dst_v,
             ba0, ba1, bb0, bb1,
             dloc, idx5, idx6, idx7, zrow_v, acc_sh, deg_sh,
             gsem_a, gsem_b, ssem):
    c = lax.axis_index("c")
    s = lax.axis_index("s")
    bufs_a = [ba0, ba1]
    bufs_b = [bb0, bb1]

    zero16 = jnp.zeros((16,), jnp.float32)
    one16 = jnp.ones((16,), jnp.float32)
    iota16 = lax.broadcasted_iota(jnp.int32, (16,), 0)
    tail_mask = iota16 >= (16 - (CHUNK - (CHUNK // 16) * 16))
    cvec = jnp.broadcast_to(c, (16,))

    stage_src = pltpu.async_copy(
        src_hbm.at[pl.ds(s * ROWS_PER_TILE, ROWS_PER_TILE)], src_v, gsem_a)
    stage_dst = pltpu.async_copy(
        dst_hbm.at[pl.ds(s * ROWS_PER_TILE, ROWS_PER_TILE)], dst_v, gsem_b)

    def fill_zrow(i, carry):
        for k in range(DH // 16):
            zrow_v[i, pl.ds(k * 16, 16)] = zero16
        return carry

    lax.fori_loop(0, ZROWS, fill_zrow, 0)

    zcopies = []
    for i in range(NODES_PER_TILE // ZROWS):
        r0 = s * NODES_PER_TILE + i * ZROWS
        zcopies.append(pltpu.async_copy(zrow_v, acc_sh.at[pl.ds(r0, ZROWS)],
                                        ssem))

    def fill_dloc(i, carry):
        dloc[i, :] = zero16
        return carry

    lax.fori_loop(0, DROWS, fill_dloc, 0)

    zcopies.append(pltpu.async_copy(
        dloc.at[pl.ds(0, DROWS_PER_TILE)],
        deg_sh.at[pl.ds(s * DROWS_PER_TILE, DROWS_PER_TILE)], ssem))

    for r in range(DROWS // 128):
        for k in range(8):
            idx5[r, pl.ds(16 * k, 16)] = iota16 + (128 * r + 16 * k)

    obase = 2 * NODES_PER_TILE * s + c
    for r in range(5):
        for k in range(7):
            idx6[r, pl.ds(16 * k, 16)] = 2 * iota16 + (2 * (125 * r + 16 * k)) + obase
        idx6[r, pl.ds(109, 16)] = 2 * iota16 + (2 * (125 * r + 109)) + obase
    for t, off in enumerate((624, 375, 384)):
        idx7[t, :] = 2 * iota16 + 2 * off + obase

    stage_src.wait()
    stage_dst.wait()

    def xform(i, carry):
        vals = [src_v[i, pl.ds(16 * k, 16)] for k in range(7)]
        vtail = src_v[i, pl.ds(109, 16)]
        for k in range(7):
            src_v[i, pl.ds(16 * k, 16)] = vals[k] + vals[k] + cvec
        src_v[i, pl.ds(109, 16)] = vtail + vtail + cvec
        return carry

    lax.fori_loop(0, ROWS_PER_TILE, xform, 0)

    def hist_row(j):
        for k in range(CHUNK // 16):
            d16 = dst_v[j, pl.ds(16 * k, 16)]
            plsc.addupdate_scatter(
                dloc, [lax.shift_right_logical(d16, 4), d16 & 15], one16)
        d16 = dst_v[j, pl.ds(CHUNK - 16, 16)]
        plsc.addupdate_scatter(
            dloc, [lax.shift_right_logical(d16, 4), d16 & 15], one16,
            mask=tail_mask)

    def do_group(g, bufs, gsem, obufs, ogsem):
        @pl.when(g + 1 < NG)
        def _():
            for b in range(K):
                pltpu.async_copy(h2_hbm.at[src_v.at[(g + 1) * K + b]],
                                 obufs[b], ogsem)

        for b in range(K):
            pltpu.make_async_copy(h2_hbm.at[src_v.at[g * K + b]],
                                  bufs[b], gsem).wait()

        scat = []
        for b in range(K):
            scat.append(pltpu.async_copy(bufs[b], acc_sh.at[dst_v.at[g * K + b]],
                                         ssem, add=True))

        for b in range(K):
            hist_row(g * K + b)

        for x in scat:
            x.wait()

    for z in zcopies:
        z.wait()
    plsc.subcore_barrier()

    for b in range(K):
        pltpu.async_copy(h2_hbm.at[src_v.at[b]], bufs_a[b], gsem_a)

    def outer(t, carry):
        do_group(2 * t, bufs_a, gsem_a, bufs_b, gsem_b)
        do_group(2 * t + 1, bufs_b, gsem_b, bufs_a, gsem_a)
        return carry

    lax.fori_loop(0, NG // 2, outer, 0)

    for r in range(DROWS // 128):
        pltpu.sync_copy(dloc.at[pl.ds(128 * r, 128)],
                        deg_sh.at[idx5.at[r]], add=True)

    plsc.subcore_barrier()

    pltpu.sync_copy(deg_sh.at[pl.ds(s * DROWS_PER_TILE, DROWS_PER_TILE)],
                    dloc.at[pl.ds(0, DROWS_PER_TILE)])

    def expand(i, carry):
        r = DROWS_PER_TILE - 1 - i
        v = 1.0 / jnp.maximum(dloc[r, :], 1.0)
        for lane in range(16):
            dloc[16 * r + lane, :] = jnp.broadcast_to(v[lane], (16,))
        return carry

    lax.fori_loop(0, DROWS_PER_TILE, expand, 0)

    def mul_rows(buf, o, length):
        def mul_row(j, carry):
            iv = dloc[o + j, :]
            for k in range(DH // 16):
                buf[j, pl.ds(16 * k, 16)] = buf[j, pl.ds(16 * k, 16)] * iv
            return carry

        lax.fori_loop(0, length, mul_row, 0)

    def stage_in(buf, o, length, sem):
        return pltpu.async_copy(
            acc_sh.at[pl.ds(s * NODES_PER_TILE + o, length)],
            buf.at[pl.ds(0, length)], sem)

    def flush(chunks):
        outs = {}
        cin = stage_in(bufs_a[0], chunks[0][0], chunks[0][1], gsem_a)
        for t, (o, length, idxref, irow) in enumerate(chunks):
            buf = (bufs_a if t % 2 == 0 else bufs_b)[0]
            cin.wait()
            if t + 1 < len(chunks):
                nbuf = (bufs_a if t % 2 == 1 else bufs_b)[0]
                nsem = gsem_a if t % 2 == 1 else gsem_b
                if t - 1 in outs:
                    outs[t - 1].wait()
                cin = stage_in(nbuf, chunks[t + 1][0], chunks[t + 1][1], nsem)
            mul_rows(buf, o, length)
            outs[t] = pltpu.async_copy(buf.at[pl.ds(0, length)],
                                       out_hbm.at[idxref.at[irow]], ssem)
        for t in (len(chunks) - 2, len(chunks) - 1):
            if t in outs:
                outs[t].wait()

    @pl.when(s < NS - 1)
    def _():
        flush([(125 * r, 125, idx6, r) for r in range(5)] + [(624, 16, idx7, 0)])

    @pl.when(s == NS - 1)
    def _():
        flush([(125 * r, 125, idx6, r) for r in range(3)]
              + [(375, 16, idx7, 1), (384, 16, idx7, 2)])


def kernel(h, edge_index):
    ei = edge_index.astype(jnp.int32)
    src = ei[0].reshape(N_EDGES // CHUNK, CHUNK)
    dst = ei[1].reshape(N_EDGES // CHUNK, CHUNK)
    h2 = h.reshape(NC * N_NODES, DH)

    mesh = plsc.VectorSubcoreMesh(core_axis_name="c", subcore_axis_name="s")

    sc_call = pl.kernel(
        _sc_body,
        mesh=mesh,
        compiler_params=pltpu.CompilerParams(use_tc_tiling_on_sc=False,
                                             needs_layout_passes=False),
        out_type=[
            jax.ShapeDtypeStruct((NC * N_NODES, DH), jnp.float32),
        ],
        scratch_types=[
            pltpu.VMEM((ROWS_PER_TILE, CHUNK), jnp.int32),
            pltpu.VMEM((ROWS_PER_TILE, CHUNK), jnp.int32),
        ] + [pltpu.VMEM((CHUNK, DH), jnp.float32)] * (2 * K) + [
            pltpu.VMEM((N_PAD // 16, 16), jnp.float32),
            pltpu.VMEM((N_PAD // 16 // 128, 128), jnp.int32),
            pltpu.VMEM((5, CHUNK), jnp.int32),
            pltpu.VMEM((3, 16), jnp.int32),
            pltpu.VMEM((ZROWS, DH), jnp.float32),
            pltpu.VMEM_SHARED((N_PAD, DH), jnp.float32),
            pltpu.VMEM_SHARED((N_PAD // 16, 16), jnp.float32),
            pltpu.SemaphoreType.DMA,
            pltpu.SemaphoreType.DMA,
            pltpu.SemaphoreType.DMA,
        ],
    )
    (out2,) = sc_call(h2, src, dst)
    return out2.reshape(N_NODES, D_FEAT)


# speedup vs baseline: 16.7399x; 1.0059x over previous
import jax
import jax.numpy as jnp
from jax import lax
from jax.experimental import pallas as pl
from jax.experimental.pallas import tpu as pltpu
from jax.experimental.pallas import tpu_sc as plsc

N_NODES = 10000
N_EDGES = 320000
D_FEAT = 128

NC = 2
NS = 16
DH = D_FEAT // NC
CHUNK = 125
ROWS_PER_TILE = N_EDGES // (NS * CHUNK)
N_PAD = 10240
NODES_PER_TILE = N_PAD // NS
ZROWS = 32
K = 2
NG = ROWS_PER_TILE // K
DROWS = N_PAD // 16
DROWS_PER_TILE = DROWS // NS


def _sc_body(h2_hbm, src_hbm, dst_hbm, out_hbm,
             src_v, dst_v,
             ba0, ba1, bb0, bb1,
             dloc, idx5, idx6, idx7, zrow_v, acc_sh, deg_sh,
             gsem_a, gsem_b, ssem):
    c = lax.axis_index("c")
    s = lax.axis_index("s")
    bufs_a = [ba0, ba1]
    bufs_b = [bb0, bb1]

    zero16 = jnp.zeros((16,), jnp.float32)
    one16 = jnp.ones((16,), jnp.float32)
    iota16 = lax.broadcasted_iota(jnp.int32, (16,), 0)
    tail_mask = iota16 >= (16 - (CHUNK - (CHUNK // 16) * 16))
    cvec = jnp.broadcast_to(c, (16,))

    stage_src = pltpu.async_copy(
        src_hbm.at[pl.ds(s * ROWS_PER_TILE, ROWS_PER_TILE)], src_v, gsem_a)
    stage_dst = pltpu.async_copy(
        dst_hbm.at[pl.ds(s * ROWS_PER_TILE, ROWS_PER_TILE)], dst_v, gsem_b)

    def fill_zrow(i, carry):
        for k in range(DH // 16):
            zrow_v[i, pl.ds(k * 16, 16)] = zero16
        return carry

    lax.fori_loop(0, ZROWS, fill_zrow, 0)

    zcopies = []
    for i in range(NODES_PER_TILE // ZROWS):
        r0 = s * NODES_PER_TILE + i * ZROWS
        zcopies.append(pltpu.async_copy(zrow_v, acc_sh.at[pl.ds(r0, ZROWS)],
                                        ssem))

    def fill_dloc(i, carry):
        dloc[i, :] = zero16
        return carry

    lax.fori_loop(0, DROWS, fill_dloc, 0)

    zcopies.append(pltpu.async_copy(
        dloc.at[pl.ds(0, DROWS_PER_TILE)],
        deg_sh.at[pl.ds(s * DROWS_PER_TILE, DROWS_PER_TILE)], ssem))

    for r in range(DROWS // 128):
        for k in range(8):
            idx5[r, pl.ds(16 * k, 16)] = iota16 + (128 * r + 16 * k)

    obase = 2 * NODES_PER_TILE * s + c
    for r in range(5):
        for k in range(7):
            idx6[r, pl.ds(16 * k, 16)] = 2 * iota16 + (2 * (125 * r + 16 * k)) + obase
        idx6[r, pl.ds(109, 16)] = 2 * iota16 + (2 * (125 * r + 109)) + obase
    for t, off in enumerate((624, 375, 384)):
        idx7[t, :] = 2 * iota16 + 2 * off + obase

    def xform_row(i):
        vals = [src_v[i, pl.ds(16 * k, 16)] for k in range(7)]
        vtail = src_v[i, pl.ds(109, 16)]
        for k in range(7):
            src_v[i, pl.ds(16 * k, 16)] = vals[k] + vals[k] + cvec
        src_v[i, pl.ds(109, 16)] = vtail + vtail + cvec

    stage_src.wait()
    stage_dst.wait()
    for b in range(K):
        xform_row(b)

    def hist_row(j):
        for k in range(CHUNK // 16):
            d16 = dst_v[j, pl.ds(16 * k, 16)]
            plsc.addupdate_scatter(
                dloc, [lax.shift_right_logical(d16, 4), d16 & 15], one16)
        d16 = dst_v[j, pl.ds(CHUNK - 16, 16)]
        plsc.addupdate_scatter(
            dloc, [lax.shift_right_logical(d16, 4), d16 & 15], one16,
            mask=tail_mask)

    def do_group(g, bufs, gsem, obufs, ogsem):
        @pl.when(g + 1 < NG)
        def _():
            for b in range(K):
                xform_row((g + 1) * K + b)
                pltpu.async_copy(h2_hbm.at[src_v.at[(g + 1) * K + b]],
                                 obufs[b], ogsem)

        for b in range(K):
            pltpu.make_async_copy(h2_hbm.at[src_v.at[g * K + b]],
                                  bufs[b], gsem).wait()

        scat = []
        for b in range(K):
            scat.append(pltpu.async_copy(bufs[b], acc_sh.at[dst_v.at[g * K + b]],
                                         ssem, add=True))

        for b in range(K):
            hist_row(g * K + b)

        for x in scat:
            x.wait()

    for z in zcopies:
        z.wait()
    plsc.subcore_barrier()

    for b in range(K):
        pltpu.async_copy(h2_hbm.at[src_v.at[b]], bufs_a[b], gsem_a)

    def outer(t, carry):
        do_group(2 * t, bufs_a, gsem_a, bufs_b, gsem_b)
        do_group(2 * t + 1, bufs_b, gsem_b, bufs_a, gsem_a)
        return carry

    lax.fori_loop(0, NG // 2, outer, 0)

    hcopies = [pltpu.async_copy(dloc.at[pl.ds(128 * r, 128)],
                                deg_sh.at[idx5.at[r]], ssem, add=True)
               for r in range(DROWS // 128)]
    for hcp in hcopies:
        hcp.wait()

    plsc.subcore_barrier()

    pltpu.sync_copy(deg_sh.at[pl.ds(s * DROWS_PER_TILE, DROWS_PER_TILE)],
                    dloc.at[pl.ds(0, DROWS_PER_TILE)])

    def expand(i, carry):
        r = DROWS_PER_TILE - 1 - i
        v = 1.0 / jnp.maximum(dloc[r, :], 1.0)
        for lane in range(16):
            dloc[16 * r + lane, :] = jnp.broadcast_to(v[lane], (16,))
        return carry

    lax.fori_loop(0, DROWS_PER_TILE, expand, 0)

    def mul_rows(buf, o, length):
        def mul_row(j, carry):
            iv = dloc[o + j, :]
            for k in range(DH // 16):
                buf[j, pl.ds(16 * k, 16)] = buf[j, pl.ds(16 * k, 16)] * iv
            return carry

        lax.fori_loop(0, length, mul_row, 0)

    def stage_in(buf, o, length, sem):
        return pltpu.async_copy(
            acc_sh.at[pl.ds(s * NODES_PER_TILE + o, length)],
            buf.at[pl.ds(0, length)], sem)

    def flush(chunks):
        outs = {}
        cin = stage_in(bufs_a[0], chunks[0][0], chunks[0][1], gsem_a)
        for t, (o, length, idxref, irow) in enumerate(chunks):
            buf = (bufs_a if t % 2 == 0 else bufs_b)[0]
            cin.wait()
            if t + 1 < len(chunks):
                nbuf = (bufs_a if t % 2 == 1 else bufs_b)[0]
                nsem = gsem_a if t % 2 == 1 else gsem_b
                if t - 1 in outs:
                    outs[t - 1].wait()
                cin = stage_in(nbuf, chunks[t + 1][0], chunks[t + 1][1], nsem)
            mul_rows(buf, o, length)
            outs[t] = pltpu.async_copy(buf.at[pl.ds(0, length)],
                                       out_hbm.at[idxref.at[irow]], ssem)
        for t in (len(chunks) - 2, len(chunks) - 1):
            if t in outs:
                outs[t].wait()

    @pl.when(s < NS - 1)
    def _():
        flush([(125 * r, 125, idx6, r) for r in range(5)] + [(624, 16, idx7, 0)])

    @pl.when(s == NS - 1)
    def _():
        flush([(125 * r, 125, idx6, r) for r in range(3)]
              + [(375, 16, idx7, 1), (384, 16, idx7, 2)])


def kernel(h, edge_index):
    ei = edge_index.astype(jnp.int32)
    src = ei[0].reshape(N_EDGES // CHUNK, CHUNK)
    dst = ei[1].reshape(N_EDGES // CHUNK, CHUNK)
    h2 = h.reshape(NC * N_NODES, DH)

    mesh = plsc.VectorSubcoreMesh(core_axis_name="c", subcore_axis_name="s")

    sc_call = pl.kernel(
        _sc_body,
        mesh=mesh,
        compiler_params=pltpu.CompilerParams(use_tc_tiling_on_sc=False,
                                             needs_layout_passes=False),
        out_type=[
            jax.ShapeDtypeStruct((NC * N_NODES, DH), jnp.float32),
        ],
        scratch_types=[
            pltpu.VMEM((ROWS_PER_TILE, CHUNK), jnp.int32),
            pltpu.VMEM((ROWS_PER_TILE, CHUNK), jnp.int32),
        ] + [pltpu.VMEM((CHUNK, DH), jnp.float32)] * (2 * K) + [
            pltpu.VMEM((N_PAD // 16, 16), jnp.float32),
            pltpu.VMEM((N_PAD // 16 // 128, 128), jnp.int32),
            pltpu.VMEM((5, CHUNK), jnp.int32),
            pltpu.VMEM((3, 16), jnp.int32),
            pltpu.VMEM((ZROWS, DH), jnp.float32),
            pltpu.VMEM_SHARED((N_PAD, DH), jnp.float32),
            pltpu.VMEM_SHARED((N_PAD // 16, 16), jnp.float32),
            pltpu.SemaphoreType.DMA,
            pltpu.SemaphoreType.DMA,
            pltpu.SemaphoreType.DMA,
        ],
    )
    (out2,) = sc_call(h2, src, dst)
    return out2.reshape(N_NODES, D_FEAT)
